# Initial kernel scaffold; baseline (speedup 1.0000x reference)
import jax
import jax.numpy as jnp
from jax.experimental import pallas as pl


def kernel(x, edge_index, W1, b1, W2, b2, W_lin, b_lin):
    raise NotImplementedError("write your pallas kernel here")


# speedup vs baseline: 14.1510x; 14.1510x over previous
import jax
import jax.numpy as jnp
from jax import lax
from jax.experimental import pallas as pl
from jax.experimental.pallas import tpu as pltpu
from jax.experimental.pallas import tpu_sc as plsc

N = 10000
NP = 10240
E = 320000
D = 128
NC = 2
NS = 16
NW = NC * NS
K = 80
EPW = E // NW
CH = 2000
NPT = NP // NS

_mesh = plsc.VectorSubcoreMesh(core_axis_name="c", subcore_axis_name="s")
_sc_params = pltpu.CompilerParams(needs_layout_passes=False)


def _zero_1d(ref, n):
    zf = jnp.zeros((16,), jnp.float32)

    @pl.loop(0, n, step=16)
    def _(i):
        ref[pl.ds(i, 16)] = zf


def _sc_degree(dst):
    @pl.kernel(
        out_type=jax.ShapeDtypeStruct((NW * NP,), jnp.float32),
        mesh=_mesh,
        compiler_params=_sc_params,
        scratch_types=[
            pltpu.VMEM((EPW,), jnp.int32),
            pltpu.VMEM((NP,), jnp.float32),
        ],
    )
    def k(dst_hbm, deg_out, dstbuf, hist):
        c = lax.axis_index("c")
        s = lax.axis_index("s")
        w = c * NS + s
        pltpu.sync_copy(dst_hbm.at[pl.ds(w * EPW, EPW)], dstbuf)
        _zero_1d(hist, NP)
        ones = jnp.full((16,), 1.0, jnp.float32)

        @pl.loop(0, EPW, step=16)
        def _(i):
            idx = dstbuf[pl.ds(i, 16)]
            plsc.addupdate_scatter(hist, [idx], ones)

        pltpu.sync_copy(hist, deg_out.at[pl.ds(w * NP, NP)])

    return k(dst)


def _tc_prep(deg3, xp, W1):
    BLK = 256
    grid = NP // BLK

    def body(deg_ref, x_ref, w1_ref, dinv_ref, y_ref):
        i = pl.program_id(0)
        deg = jnp.sum(deg_ref[...], axis=0) + 1.0
        rows = i * BLK + lax.broadcasted_iota(jnp.int32, (BLK, 1), 0)
        dinv = jnp.where(rows < N, lax.rsqrt(deg), 0.0)
        dinv_ref[...] = dinv
        y_ref[...] = (
            jnp.dot(x_ref[...], w1_ref[...], preferred_element_type=jnp.float32)
            * dinv
        )

    return pl.pallas_call(
        body,
        grid=(grid,),
        in_specs=[
            pl.BlockSpec((NW, BLK, 1), lambda i: (0, i, 0)),
            pl.BlockSpec((BLK, D), lambda i: (i, 0)),
            pl.BlockSpec((D, D), lambda i: (0, 0)),
        ],
        out_specs=[
            pl.BlockSpec((BLK, 1), lambda i: (i, 0)),
            pl.BlockSpec((BLK, D), lambda i: (i, 0)),
        ],
        out_shape=[
            jax.ShapeDtypeStruct((NP, 1), jnp.float32),
            jax.ShapeDtypeStruct((NP, D), jnp.float32),
        ],
    )(deg3, xp, W1)


def _sc_agg(src, dst, dinv, y):
    @pl.kernel(
        out_type=[
            jax.ShapeDtypeStruct((NW * NP,), jnp.float32),
            jax.ShapeDtypeStruct((NC, NP, D), jnp.float32),
        ],
        mesh=_mesh,
        compiler_params=_sc_params,
        scratch_types=[
            pltpu.VMEM((CH,), jnp.int32),
            pltpu.VMEM((CH,), jnp.int32),
            pltpu.VMEM((NP,), jnp.float32),
            pltpu.VMEM((NP,), jnp.float32),
            pltpu.VMEM((K, D), jnp.float32),
            pltpu.VMEM_SHARED((NP, D), jnp.float32),
            pltpu.SemaphoreType.DMA,
        ],
    )
    def k(src_hbm, dst_hbm, dinv_hbm, y_hbm, s_out, agg_out,
          srcbuf, dstbuf, dinvbuf, sbuf, rows0, acc, sem):
        c = lax.axis_index("c")
        s = lax.axis_index("s")
        w = c * NS + s
        pltpu.sync_copy(dinv_hbm, dinvbuf)
        _zero_1d(sbuf, NP)

        zf = jnp.zeros((16,), jnp.float32)

        @pl.loop(0, K)
        def _(r):
            @pl.loop(0, D, step=16)
            def _(j):
                rows0[r, pl.ds(j, 16)] = zf

        @pl.loop(0, NPT // K)
        def _(t):
            pltpu.sync_copy(rows0, acc.at[pl.ds(s * NPT + t * K, K)])

        plsc.subcore_barrier()

        @pl.loop(0, EPW, step=CH)
        def _(e0):
            pltpu.sync_copy(src_hbm.at[pl.ds(w * EPW + e0, CH)], srcbuf)
            pltpu.sync_copy(dst_hbm.at[pl.ds(w * EPW + e0, CH)], dstbuf)

            @pl.loop(0, CH, step=16)
            def _(i):
                didx = dstbuf[pl.ds(i, 16)]
                vals = plsc.load_gather(dinvbuf, [didx])
                sidx = srcbuf[pl.ds(i, 16)]
                plsc.addupdate_scatter(sbuf, [sidx], vals)

            @pl.loop(0, CH, step=K)
            def _(r):
                pltpu.async_copy(
                    y_hbm.at[srcbuf.at[pl.ds(r, K)]], rows0, sem
                ).wait()
                pltpu.sync_copy(
                    rows0, acc.at[dstbuf.at[pl.ds(r, K)]], add=True
                )

        pltpu.sync_copy(sbuf, s_out.at[pl.ds(w * NP, NP)])
        plsc.subcore_barrier()
        pltpu.sync_copy(
            acc.at[pl.ds(s * NPT, NPT)], agg_out.at[c, pl.ds(s * NPT, NPT)]
        )

    return k(src, dst, dinv, y)


def _tc_final(agg, y, dinv2, sp3, b1, W2, b2, WlT, bl):
    BLK = 256
    grid = NP // BLK

    def body(agg_ref, y_ref, dinv_ref, sp_ref, b1_ref, w2_ref, b2_ref,
             wlt_ref, bl_ref, out_ref, acc_ref):
        i = pl.program_id(0)

        @pl.when(i == 0)
        def _():
            acc_ref[...] = jnp.zeros_like(acc_ref)

        dinv = dinv_ref[...]
        aggs = agg_ref[0] + agg_ref[1] + y_ref[...]
        h = dinv * aggs + b1_ref[...]
        h = jnp.where(h > 0, h, 0.01 * h)
        sv = jnp.sum(sp_ref[...], axis=0)
        wgt = dinv * sv + dinv * dinv
        acc_ref[...] += jnp.sum(wgt * h, axis=0, keepdims=True)

        @pl.when(i == pl.num_programs(0) - 1)
        def _():
            pooled = acc_ref[...]
            t = (
                jnp.dot(pooled, w2_ref[...], preferred_element_type=jnp.float32)
                + N * b2_ref[...]
            )
            out_ref[...] = (
                jnp.dot(t, wlt_ref[...], preferred_element_type=jnp.float32)
                + bl_ref[...]
            )

    return pl.pallas_call(
        body,
        grid=(grid,),
        in_specs=[
            pl.BlockSpec((NC, BLK, D), lambda i: (0, i, 0)),
            pl.BlockSpec((BLK, D), lambda i: (i, 0)),
            pl.BlockSpec((BLK, 1), lambda i: (i, 0)),
            pl.BlockSpec((NW, BLK, 1), lambda i: (0, i, 0)),
            pl.BlockSpec((1, D), lambda i: (0, 0)),
            pl.BlockSpec((D, D), lambda i: (0, 0)),
            pl.BlockSpec((1, D), lambda i: (0, 0)),
            pl.BlockSpec((D, 2), lambda i: (0, 0)),
            pl.BlockSpec((1, 2), lambda i: (0, 0)),
        ],
        out_specs=pl.BlockSpec((1, 2), lambda i: (0, 0)),
        out_shape=jax.ShapeDtypeStruct((1, 2), jnp.float32),
        scratch_shapes=[pltpu.VMEM((1, D), jnp.float32)],
    )(agg, y, dinv2, sp3, b1, W2, b2, WlT, bl)


def kernel(x, edge_index, W1, b1, W2, b2, W_lin, b_lin):
    src = edge_index[0].astype(jnp.int32)
    dst = edge_index[1].astype(jnp.int32)

    deg_part = _sc_degree(dst)
    xp = jnp.pad(x, ((0, NP - N), (0, 0)))
    dinv2, y = _tc_prep(deg_part.reshape(NW, NP, 1), xp, W1)
    dinv_flat = dinv2.reshape(NP)

    s_part, agg = _sc_agg(src, dst, dinv_flat, y)

    return _tc_final(
        agg, y, dinv2, s_part.reshape(NW, NP, 1),
        b1.reshape(1, D), W2, b2.reshape(1, D), W_lin.T, b_lin.reshape(1, 2),
    )
